# baseline (speedup 1.0000x reference)
import functools

import jax
import jax.numpy as jnp
import numpy as np
from jax import lax
from jax.experimental import pallas as pl
from jax.experimental.pallas import tpu as pltpu
from jax.experimental.pallas import tpu_sc as plsc

N_NODES = 100000
N_EDGES = 1600000
B = 2048
K = 20
D = 128
D_TIME = 128
D_EDGE = 16
N_HEADS = 2
DH = D // N_HEADS

NC = 2
NS = 16
NW = NC * NS
BK = B * K
ROWS_W = BK // NW
SRC_W = B // NW
CH = 128
NCH = ROWS_W // CH


NBUF = 4
ECH = 160
ENCH = ROWS_W // ECH
EG = ECH // 16


def _sc_edges(et3_hbm, eidx_v, ef2_out, slabA, slabB, eb0, eb1,
              semA, semB, wse0, wse1, wid):
    ebufs = [eb0, eb1]
    ewr = [None, None]

    def _scalar_lane(vec, s):
        return vec[s]

    def _issue(c, g, slab, sem):
        idxc = eidx_v[c, pl.ds(g * 16, 16)]
        for s in range(16):
            e = _scalar_lane(idxc, s)
            off = pl.multiple_of((e // 8) * 8, 8)
            pltpu.async_copy(et3_hbm.at[pl.ds(off, 8)], slab.at[pl.ds(s * 8, 8)], sem)

    def _extract(c, g, slab, sem, ebuf):
        for s in range(16):
            pltpu.make_async_copy(et3_hbm.at[pl.ds(0, 8)], slab.at[pl.ds(s * 8, 8)], sem).wait()
        idxp = eidx_v[c, pl.ds(g * 16, 16)]
        for s in range(16):
            e = _scalar_lane(idxp, s)
            lo = lax.rem(e, 8)
            j = g * 16 + s
            r = j // K
            colk = lax.rem(j, K) * D_EDGE
            ebuf[r, pl.ds(colk, D_EDGE)] = slab[s * 8 + lo, pl.ds(0, D_EDGE)]

    for c in range(ENCH):
        ebuf = ebufs[c % 2]
        if ewr[c % 2] is not None:
            ewr[c % 2].wait()

        def _pair(p, carry):
            _issue(c, 2 * p, slabA, semA)

            @pl.when(p >= 1)
            def _():
                _extract(c, 2 * p - 1, slabB, semB, ebuf)

            _issue(c, 2 * p + 1, slabB, semB)
            _extract(c, 2 * p, slabA, semA, ebuf)
            return carry

        lax.fori_loop(0, EG // 2, _pair, 0, unroll=False)
        _extract(c, EG - 1, slabB, semB, ebuf)
        ewr[c % 2] = pltpu.async_copy(
            ebuf, ef2_out.at[pl.ds(wid * SRC_W + c * 8, 8)],
            wse0 if c % 2 == 0 else wse1)
    ewr[0].wait()
    ewr[1].wait()


def _sc_gather_body(mem_hbm, nf_hbm, sidx_hbm, nidx_hbm, et3_hbm, eidx_hbm,
                    src_out, nbr_out, ef2_out,
                    sidx_v, nidx_v, eidx_v, b0, b1, b2, b3, sa_v,
                    slabA, slabB, eb0, eb1,
                    sem_g, sem_h, sem_s, w0, w1, w2, w3,
                    semA, semB, wse0, wse1):
    wid = lax.axis_index("s") * NC + lax.axis_index("c")
    nbase = wid * ROWS_W
    bufs = [b0, b1, b2, b3]
    wsems = [w0, w1, w2, w3]

    pltpu.sync_copy(nidx_hbm.at[wid], nidx_v)
    pltpu.sync_copy(eidx_hbm.at[wid], eidx_v)
    pltpu.sync_copy(sidx_hbm.at[pl.ds(wid * SRC_W, SRC_W)], sidx_v)

    pltpu.async_copy(mem_hbm.at[sidx_v], sa_v, sem_s).wait()
    pltpu.async_copy(nf_hbm.at[sidx_v], sa_v, sem_s, add=True).wait()
    w_src = pltpu.async_copy(sa_v, src_out.at[pl.ds(wid * SRC_W, SRC_W)], sem_s)

    g1 = [None] * NCH
    g2 = [None] * NCH
    wr = [None] * NCH
    for c in range(min(NBUF, NCH)):
        g1[c] = pltpu.async_copy(mem_hbm.at[nidx_v.at[c]], bufs[c % NBUF], sem_g)
    for c in range(NCH):
        buf = bufs[c % NBUF]
        g1[c].wait()
        g2[c] = pltpu.async_copy(nf_hbm.at[nidx_v.at[c]], buf, sem_h, add=True)
        if c >= 1 and c - 1 + NBUF < NCH:
            nxt = c - 1 + NBUF
            wr[c - 1].wait()
            g1[nxt] = pltpu.async_copy(
                mem_hbm.at[nidx_v.at[nxt]], bufs[nxt % NBUF], sem_g)
        g2[c].wait()
        wr[c] = pltpu.async_copy(
            buf, nbr_out.at[pl.ds(nbase + c * CH, CH)], wsems[c % NBUF])

    _sc_edges(et3_hbm, eidx_v, ef2_out, slabA, slabB, eb0, eb1,
              semA, semB, wse0, wse1, wid)

    for c in range(max(0, NCH - NBUF), NCH):
        wr[c].wait()
    w_src.wait()


def _sc_gather(memory, node_features, src_idx, nbr_idx3, et3, eidx3):
    mesh = plsc.VectorSubcoreMesh(core_axis_name="c", subcore_axis_name="s")
    fn = pl.kernel(
        _sc_gather_body,
        mesh=mesh,
        out_type=(
            jax.ShapeDtypeStruct((B, D), jnp.float32),
            jax.ShapeDtypeStruct((BK, D), jnp.float32),
            jax.ShapeDtypeStruct((B, K * D_EDGE), jnp.float32),
        ),
        scratch_types=[
            pltpu.VMEM((SRC_W,), jnp.int32),
            pltpu.VMEM((NCH, CH), jnp.int32),
            pltpu.VMEM((ENCH, ECH), jnp.int32),
            pltpu.VMEM((CH, D), jnp.float32),
            pltpu.VMEM((CH, D), jnp.float32),
            pltpu.VMEM((CH, D), jnp.float32),
            pltpu.VMEM((CH, D), jnp.float32),
            pltpu.VMEM((SRC_W, D), jnp.float32),
            pltpu.VMEM((16 * 8, D_EDGE), jnp.float32),
            pltpu.VMEM((16 * 8, D_EDGE), jnp.float32),
            pltpu.VMEM((8, K * D_EDGE), jnp.float32),
            pltpu.VMEM((8, K * D_EDGE), jnp.float32),
            pltpu.SemaphoreType.DMA,
            pltpu.SemaphoreType.DMA,
            pltpu.SemaphoreType.DMA,
            pltpu.SemaphoreType.DMA,
            pltpu.SemaphoreType.DMA,
            pltpu.SemaphoreType.DMA,
            pltpu.SemaphoreType.DMA,
            pltpu.SemaphoreType.DMA,
            pltpu.SemaphoreType.DMA,
            pltpu.SemaphoreType.DMA,
            pltpu.SemaphoreType.DMA,
        ],
    )
    return fn(memory, node_features, src_idx, nbr_idx3, et3, eidx3)


BB = 256
_PREC = lax.Precision.DEFAULT

_INV2PI = np.float32(1.0 / (2.0 * np.pi))
_RBIG = np.float32(12582912.0)
_C1 = np.float32(6.28125)
_C2 = np.float32(2.0 * np.pi - 6.28125)
_C3 = np.float32(2.0 * np.pi - 6.28125 - float(np.float32(2.0 * np.pi - 6.28125)))
_COS_COEF = tuple(np.float32(c) for c in (
    1.0, -0.5, 0.0416666641831398, -0.0013888857793062925,
    2.4800388928269967e-05, -2.753230603502743e-07,
    2.0584800530798475e-09, -9.666989431167394e-12))


def _vcos(x):
    n = lax.round(x * _INV2PI, lax.RoundingMethod.TO_NEAREST_EVEN)
    r = ((x - n * _C1) - n * _C2) - n * _C3
    s = r * r
    acc = jnp.full_like(s, _COS_COEF[7])
    for c in _COS_COEF[6::-1]:
        acc = acc * s + c
    return acc


_r = np.arange(K * D)
_colmap = (_r // D) + K * ((_r % D) >= DH)
_HM2 = (_colmap[:, None] == np.arange(2 * K)[None, :]).astype(np.float32)
_HM2T = np.ascontiguousarray(_HM2.T)
_ISTK = np.tile(np.eye(D, dtype=np.float32), (K, 1))
del _r, _colmap


def _tc_body(src_ref, nbr_ref, ef_ref, ts_ref, ets_ref, nid_ref, tw_ref, tb_ref,
             wq1, wq2, wk1, wk2, wk3bd, wv1, wv2, wv3bd, wo1, wo2,
             hm2, hm2t, istk, out_ref, pk_scr, vv_scr):
    src = src_ref[...]
    tw = tw_ref[...]
    tb = tb_ref[...]
    scale = np.float32(1.0 / np.sqrt(DH))
    q_const = jnp.dot(_vcos(tb), wq2[...])
    q = (jnp.dot(src, wq1[...]) + q_const) * scale
    EK = jnp.dot(ef_ref[...], wk3bd[...])
    EV = jnp.dot(ef_ref[...], wv3bd[...])
    delta_all = ts_ref[...] - ets_ref[...]

    for k in range(K):
        nbr_k = nbr_ref[k]
        dk = jnp.broadcast_to(delta_all[:, k:k + 1], (BB, D_TIME))
        te = _vcos(dk * tw + tb)
        sl = pl.ds(k * D, D)
        kk = jnp.dot(nbr_k, wk1[...]) + jnp.dot(te, wk2[...]) + EK[:, k * D:(k + 1) * D]
        vv = jnp.dot(nbr_k, wv1[...]) + jnp.dot(te, wv2[...]) + EV[:, k * D:(k + 1) * D]
        pk_scr[:, sl] = q * kk
        vv_scr[:, sl] = vv

    S = jnp.dot(pk_scr[...], hm2[...])
    padf = (nid_ref[...] == 0).astype(jnp.float32)
    mask2 = jnp.concatenate([padf, padf], axis=1)
    S = jnp.where(mask2 > 0.5, jnp.float32(-1e9), S)
    m0 = jnp.max(S[:, :K], axis=1, keepdims=True)
    m1 = jnp.max(S[:, K:], axis=1, keepdims=True)
    mb = jnp.concatenate([jnp.broadcast_to(m0, (BB, K)),
                          jnp.broadcast_to(m1, (BB, K))], axis=1)
    E = jnp.exp(S - mb)
    l0 = jnp.sum(E[:, :K], axis=1, keepdims=True)
    l1 = jnp.sum(E[:, K:], axis=1, keepdims=True)
    lb = jnp.concatenate([jnp.broadcast_to(l0, (BB, K)),
                          jnp.broadcast_to(l1, (BB, K))], axis=1)
    attn = E / lb
    W = jnp.dot(attn, hm2t[...])
    attn_out = jnp.dot(W * vv_scr[...], istk[...])
    out_ref[...] = jnp.dot(attn_out, wo1[...]) + jnp.dot(src, wo2[...])


def _tc_dense(src_conv, nbr3, ef2, ts2, ets, nids, time_w, time_b2,
              Wq1, Wq2, Wk1, Wk2, Wk3bd, Wv1, Wv2, Wv3bd, Wo1, Wo2):
    row = lambda i: (i, 0)
    fixed = lambda i: (0, 0)
    return pl.pallas_call(
        _tc_body,
        grid=(B // BB,),
        in_specs=[
            pl.BlockSpec((BB, D), row),
            pl.BlockSpec((K, BB, D), lambda i: (0, i, 0)),
            pl.BlockSpec((BB, K * D_EDGE), row),
            pl.BlockSpec((BB, 1), row),
            pl.BlockSpec((BB, K), row),
            pl.BlockSpec((BB, K), row),
            pl.BlockSpec((1, D_TIME), fixed),
            pl.BlockSpec((1, D_TIME), fixed),
            pl.BlockSpec((D, D), fixed),
            pl.BlockSpec((D_TIME, D), fixed),
            pl.BlockSpec((D, D), fixed),
            pl.BlockSpec((D_TIME, D), fixed),
            pl.BlockSpec((K * D_EDGE, K * D), fixed),
            pl.BlockSpec((D, D), fixed),
            pl.BlockSpec((D_TIME, D), fixed),
            pl.BlockSpec((K * D_EDGE, K * D), fixed),
            pl.BlockSpec((D, D), fixed),
            pl.BlockSpec((D, D), fixed),
            pl.BlockSpec((K * D, 2 * K), fixed),
            pl.BlockSpec((2 * K, K * D), fixed),
            pl.BlockSpec((K * D, D), fixed),
        ],
        out_specs=pl.BlockSpec((BB, D), row),
        out_shape=jax.ShapeDtypeStruct((B, D), jnp.float32),
        scratch_shapes=[
            pltpu.VMEM((BB, K * D), jnp.float32),
            pltpu.VMEM((BB, K * D), jnp.float32),
        ],
    )(src_conv, nbr3, ef2, ts2, ets, nids, time_w, time_b2,
      Wq1, Wq2, Wk1, Wk2, Wk3bd, Wv1, Wv2, Wv3bd, Wo1, Wo2,
      jnp.asarray(_HM2), jnp.asarray(_HM2T), jnp.asarray(_ISTK))


def kernel(memory, source_nodes, timestamps, n_layers, neighbors, edge_idxs,
           edge_times, node_features, edge_features, time_w, time_b,
           Wq, Wk, Wv, Wout):
    del n_layers
    src_idx = source_nodes.astype(jnp.int32)
    nbr_idx3 = neighbors.astype(jnp.int32).T.reshape(NW, NCH, CH)
    et3 = edge_features
    eidx3 = edge_idxs.astype(jnp.int32).reshape(NW, ENCH, ECH)

    src_conv, nbr_emb, ef2 = _sc_gather(
        memory, node_features, src_idx, nbr_idx3, et3, eidx3)
    nbr3 = nbr_emb.reshape(K, B, D)

    ts2 = timestamps.reshape(B, 1)
    time_b2 = time_b.reshape(1, D_TIME)

    Wq1, Wq2 = Wq[:D], Wq[D:]
    Wk1, Wk2, Wk3 = Wk[:D], Wk[D:D + D_TIME], Wk[D + D_TIME:]
    Wv1, Wv2, Wv3 = Wv[:D], Wv[D:D + D_TIME], Wv[D + D_TIME:]
    Wo1, Wo2 = Wout[:D], Wout[D:]
    eyeK = jnp.eye(K, dtype=jnp.float32)
    Wk3bd = jnp.kron(eyeK, Wk3)
    Wv3bd = jnp.kron(eyeK, Wv3)

    return _tc_dense(src_conv, nbr3, ef2, ts2, edge_times, neighbors.astype(jnp.int32),
                     time_w, time_b2, Wq1, Wq2, Wk1, Wk2, Wk3bd, Wv1, Wv2, Wv3bd, Wo1, Wo2)


# speedup vs baseline: 4.2035x; 4.2035x over previous
import functools

import jax
import jax.numpy as jnp
import numpy as np
from jax import lax
from jax.experimental import pallas as pl
from jax.experimental.pallas import tpu as pltpu
from jax.experimental.pallas import tpu_sc as plsc

N_NODES = 100000
N_EDGES = 1600000
B = 2048
K = 20
D = 128
D_TIME = 128
D_EDGE = 16
N_HEADS = 2
DH = D // N_HEADS

NC = 2
NS = 16
NW = NC * NS
BK = B * K
ROWS_W = BK // NW
SRC_W = B // NW
CH = 128
NCH = ROWS_W // CH


NBUF = 4
def _sc_gather_body(mem_hbm, nf_hbm, sidx_hbm, nidx_hbm,
                    src_out, nbr_out,
                    sidx_v, nidx_v, b0, b1, b2, b3, sa_v,
                    sem_g, sem_h, sem_s, w0, w1, w2, w3):
    wid = lax.axis_index("s") * NC + lax.axis_index("c")
    nbase = wid * ROWS_W
    bufs = [b0, b1, b2, b3]
    wsems = [w0, w1, w2, w3]

    pltpu.sync_copy(nidx_hbm.at[wid], nidx_v)
    pltpu.sync_copy(sidx_hbm.at[pl.ds(wid * SRC_W, SRC_W)], sidx_v)

    pltpu.async_copy(mem_hbm.at[sidx_v], sa_v, sem_s).wait()
    pltpu.async_copy(nf_hbm.at[sidx_v], sa_v, sem_s, add=True).wait()
    w_src = pltpu.async_copy(sa_v, src_out.at[pl.ds(wid * SRC_W, SRC_W)], sem_s)

    g1 = [None] * NCH
    g2 = [None] * NCH
    wr = [None] * NCH
    for c in range(min(NBUF, NCH)):
        g1[c] = pltpu.async_copy(mem_hbm.at[nidx_v.at[c]], bufs[c % NBUF], sem_g)
    for c in range(NCH):
        buf = bufs[c % NBUF]
        g1[c].wait()
        g2[c] = pltpu.async_copy(nf_hbm.at[nidx_v.at[c]], buf, sem_h, add=True)
        if c >= 1 and c - 1 + NBUF < NCH:
            nxt = c - 1 + NBUF
            wr[c - 1].wait()
            g1[nxt] = pltpu.async_copy(
                mem_hbm.at[nidx_v.at[nxt]], bufs[nxt % NBUF], sem_g)
        g2[c].wait()
        wr[c] = pltpu.async_copy(
            buf, nbr_out.at[pl.ds(nbase + c * CH, CH)], wsems[c % NBUF])


    for c in range(max(0, NCH - NBUF), NCH):
        wr[c].wait()
    w_src.wait()


def _sc_gather(memory, node_features, src_idx, nbr_idx3):
    mesh = plsc.VectorSubcoreMesh(core_axis_name="c", subcore_axis_name="s")
    fn = pl.kernel(
        _sc_gather_body,
        mesh=mesh,
        out_type=(
            jax.ShapeDtypeStruct((B, D), jnp.float32),
            jax.ShapeDtypeStruct((BK, D), jnp.float32),
        ),
        scratch_types=[
            pltpu.VMEM((SRC_W,), jnp.int32),
            pltpu.VMEM((NCH, CH), jnp.int32),
            pltpu.VMEM((CH, D), jnp.float32),
            pltpu.VMEM((CH, D), jnp.float32),
            pltpu.VMEM((CH, D), jnp.float32),
            pltpu.VMEM((CH, D), jnp.float32),
            pltpu.VMEM((SRC_W, D), jnp.float32),
            pltpu.SemaphoreType.DMA,
            pltpu.SemaphoreType.DMA,
            pltpu.SemaphoreType.DMA,
            pltpu.SemaphoreType.DMA,
            pltpu.SemaphoreType.DMA,
            pltpu.SemaphoreType.DMA,
            pltpu.SemaphoreType.DMA,
        ],
    )
    return fn(memory, node_features, src_idx, nbr_idx3)


BB = 256
_PREC = lax.Precision.DEFAULT

_INV2PI = np.float32(1.0 / (2.0 * np.pi))
_RBIG = np.float32(12582912.0)
_C1 = np.float32(6.28125)
_C2 = np.float32(2.0 * np.pi - 6.28125)
_C3 = np.float32(2.0 * np.pi - 6.28125 - float(np.float32(2.0 * np.pi - 6.28125)))
_COS_COEF = tuple(np.float32(c) for c in (
    1.0, -0.5, 0.0416666641831398, -0.0013888857793062925,
    2.4800388928269967e-05, -2.753230603502743e-07,
    2.0584800530798475e-09, -9.666989431167394e-12))


def _vcos(x):
    n = lax.round(x * _INV2PI, lax.RoundingMethod.TO_NEAREST_EVEN)
    r = ((x - n * _C1) - n * _C2) - n * _C3
    s = r * r
    acc = jnp.full_like(s, _COS_COEF[7])
    for c in _COS_COEF[6::-1]:
        acc = acc * s + c
    return acc


_r = np.arange(K * D)
_colmap = (_r // D) + K * ((_r % D) >= DH)
_HM2 = (_colmap[:, None] == np.arange(2 * K)[None, :]).astype(np.float32)
_HM2T = np.ascontiguousarray(_HM2.T)
_ISTK = np.tile(np.eye(D, dtype=np.float32), (K, 1))
del _r, _colmap


def _tc_body(src_ref, nbr_ref, ef_ref, ts_ref, ets_ref, nid_ref, tw_ref, tb_ref,
             wq1, wq2, wk1, wk2, wk3bd, wv1, wv2, wv3bd, wo1, wo2,
             hm2, hm2t, istk, out_ref, pk_scr, vv_scr):
    src = src_ref[...]
    tw = tw_ref[...]
    tb = tb_ref[...]
    scale = np.float32(1.0 / np.sqrt(DH))
    q_const = jnp.dot(_vcos(tb), wq2[...])
    q = (jnp.dot(src, wq1[...]) + q_const) * scale
    EK = jnp.dot(ef_ref[...], wk3bd[...])
    EV = jnp.dot(ef_ref[...], wv3bd[...])
    delta_all = ts_ref[...] - ets_ref[...]

    for k in range(K):
        nbr_k = nbr_ref[k]
        dk = jnp.broadcast_to(delta_all[:, k:k + 1], (BB, D_TIME))
        te = _vcos(dk * tw + tb)
        sl = pl.ds(k * D, D)
        kk = jnp.dot(nbr_k, wk1[...]) + jnp.dot(te, wk2[...]) + EK[:, k * D:(k + 1) * D]
        vv = jnp.dot(nbr_k, wv1[...]) + jnp.dot(te, wv2[...]) + EV[:, k * D:(k + 1) * D]
        pk_scr[:, sl] = q * kk
        vv_scr[:, sl] = vv

    S = jnp.dot(pk_scr[...], hm2[...])
    padf = (nid_ref[...] == 0).astype(jnp.float32)
    mask2 = jnp.concatenate([padf, padf], axis=1)
    S = jnp.where(mask2 > 0.5, jnp.float32(-1e9), S)
    m0 = jnp.max(S[:, :K], axis=1, keepdims=True)
    m1 = jnp.max(S[:, K:], axis=1, keepdims=True)
    mb = jnp.concatenate([jnp.broadcast_to(m0, (BB, K)),
                          jnp.broadcast_to(m1, (BB, K))], axis=1)
    E = jnp.exp(S - mb)
    l0 = jnp.sum(E[:, :K], axis=1, keepdims=True)
    l1 = jnp.sum(E[:, K:], axis=1, keepdims=True)
    lb = jnp.concatenate([jnp.broadcast_to(l0, (BB, K)),
                          jnp.broadcast_to(l1, (BB, K))], axis=1)
    attn = E / lb
    W = jnp.dot(attn, hm2t[...])
    attn_out = jnp.dot(W * vv_scr[...], istk[...])
    out_ref[...] = jnp.dot(attn_out, wo1[...]) + jnp.dot(src, wo2[...])


def _tc_dense(src_conv, nbr3, ef2, ts2, ets, nids, time_w, time_b2,
              Wq1, Wq2, Wk1, Wk2, Wk3bd, Wv1, Wv2, Wv3bd, Wo1, Wo2):
    row = lambda i: (i, 0)
    fixed = lambda i: (0, 0)
    return pl.pallas_call(
        _tc_body,
        grid=(B // BB,),
        in_specs=[
            pl.BlockSpec((BB, D), row),
            pl.BlockSpec((K, BB, D), lambda i: (0, i, 0)),
            pl.BlockSpec((BB, K * D_EDGE), row),
            pl.BlockSpec((BB, 1), row),
            pl.BlockSpec((BB, K), row),
            pl.BlockSpec((BB, K), row),
            pl.BlockSpec((1, D_TIME), fixed),
            pl.BlockSpec((1, D_TIME), fixed),
            pl.BlockSpec((D, D), fixed),
            pl.BlockSpec((D_TIME, D), fixed),
            pl.BlockSpec((D, D), fixed),
            pl.BlockSpec((D_TIME, D), fixed),
            pl.BlockSpec((K * D_EDGE, K * D), fixed),
            pl.BlockSpec((D, D), fixed),
            pl.BlockSpec((D_TIME, D), fixed),
            pl.BlockSpec((K * D_EDGE, K * D), fixed),
            pl.BlockSpec((D, D), fixed),
            pl.BlockSpec((D, D), fixed),
            pl.BlockSpec((K * D, 2 * K), fixed),
            pl.BlockSpec((2 * K, K * D), fixed),
            pl.BlockSpec((K * D, D), fixed),
        ],
        out_specs=pl.BlockSpec((BB, D), row),
        out_shape=jax.ShapeDtypeStruct((B, D), jnp.float32),
        scratch_shapes=[
            pltpu.VMEM((BB, K * D), jnp.float32),
            pltpu.VMEM((BB, K * D), jnp.float32),
        ],
    )(src_conv, nbr3, ef2, ts2, ets, nids, time_w, time_b2,
      Wq1, Wq2, Wk1, Wk2, Wk3bd, Wv1, Wv2, Wv3bd, Wo1, Wo2,
      jnp.asarray(_HM2), jnp.asarray(_HM2T), jnp.asarray(_ISTK))


def kernel(memory, source_nodes, timestamps, n_layers, neighbors, edge_idxs,
           edge_times, node_features, edge_features, time_w, time_b,
           Wq, Wk, Wv, Wout):
    del n_layers
    src_idx = source_nodes.astype(jnp.int32)
    nbr_idx3 = neighbors.astype(jnp.int32).T.reshape(NW, NCH, CH)
    src_conv, nbr_emb = _sc_gather(memory, node_features, src_idx, nbr_idx3)
    nbr3 = nbr_emb.reshape(K, B, D)

    ef2 = jnp.take(edge_features, edge_idxs.reshape(-1), axis=0).reshape(B, K * D_EDGE)

    ts2 = timestamps.reshape(B, 1)
    time_b2 = time_b.reshape(1, D_TIME)

    Wq1, Wq2 = Wq[:D], Wq[D:]
    Wk1, Wk2, Wk3 = Wk[:D], Wk[D:D + D_TIME], Wk[D + D_TIME:]
    Wv1, Wv2, Wv3 = Wv[:D], Wv[D:D + D_TIME], Wv[D + D_TIME:]
    Wo1, Wo2 = Wout[:D], Wout[D:]
    eyeK = jnp.eye(K, dtype=jnp.float32)
    Wk3bd = jnp.kron(eyeK, Wk3)
    Wv3bd = jnp.kron(eyeK, Wv3)

    return _tc_dense(src_conv, nbr3, ef2, ts2, edge_times, neighbors.astype(jnp.int32),
                     time_w, time_b2, Wq1, Wq2, Wk1, Wk2, Wk3bd, Wv1, Wv2, Wv3bd, Wo1, Wo2)
